# Initial kernel scaffold; baseline (speedup 1.0000x reference)
import jax
import jax.numpy as jnp
from jax.experimental import pallas as pl


def kernel(x, edge_index, W1_l, b1, W1_r, W2_l, b2, W2_r):
    raise NotImplementedError("write your pallas kernel here")


# speedup vs baseline: 1.0055x; 1.0055x over previous
import functools

import jax
import jax.numpy as jnp
from jax.experimental import pallas as pl
from jax.experimental.pallas import tpu as pltpu

N = 10000
E = 160000
D = 256
BM = 1000


def _combine_body(s_ref, cnt_ref, x_ref, wlt_ref, b_ref, wrt_ref, o_ref, *, relu):
    inv = 1.0 / jnp.maximum(cnt_ref[...], 1.0)
    mean = s_ref[...] * inv
    acc = jnp.dot(mean, wlt_ref[...], preferred_element_type=jnp.float32)
    acc = acc + jnp.dot(x_ref[...], wrt_ref[...], preferred_element_type=jnp.float32)
    acc = acc + b_ref[...]
    o_ref[...] = jnp.maximum(acc, 0.0) if relu else acc


def _combine(s, cnt, x, wlt, b, wrt, relu):
    grid = (N // BM,)
    return pl.pallas_call(
        functools.partial(_combine_body, relu=relu),
        grid=grid,
        in_specs=[
            pl.BlockSpec((BM, D), lambda i: (i, 0)),
            pl.BlockSpec((BM, 1), lambda i: (i, 0)),
            pl.BlockSpec((BM, D), lambda i: (i, 0)),
            pl.BlockSpec((D, D), lambda i: (0, 0)),
            pl.BlockSpec((1, D), lambda i: (0, 0)),
            pl.BlockSpec((D, D), lambda i: (0, 0)),
        ],
        out_specs=pl.BlockSpec((BM, D), lambda i: (i, 0)),
        out_shape=jax.ShapeDtypeStruct((N, D), jnp.float32),
    )(s, cnt, x, wlt, b, wrt)


def kernel(x, edge_index, W1_l, b1, W1_r, W2_l, b2, W2_r):
    src = edge_index[0]
    dst = edge_index[1]
    cnt = jax.ops.segment_sum(jnp.ones((E,), jnp.float32), dst, num_segments=N)
    cnt = cnt.reshape(N, 1)

    def seg_mean_sum(h):
        return jax.ops.segment_sum(jnp.take(h, src, axis=0), dst, num_segments=N)

    s1 = seg_mean_sum(x)
    h = _combine(s1, cnt, x, W1_l.T, b1.reshape(1, D), W1_r.T, relu=True)
    s2 = seg_mean_sum(h)
    out = _combine(s2, cnt, h, W2_l.T, b2.reshape(1, D), W2_r.T, relu=False)
    return out


# speedup vs baseline: 2.6603x; 2.6457x over previous
import functools

import jax
import jax.numpy as jnp
from jax import lax
from jax.experimental import pallas as pl
from jax.experimental.pallas import tpu as pltpu
from jax.experimental.pallas import tpu_sc as plsc

N = 10000
E = 160000
D = 256
DH = 128
NCORE = 2
NSUB = 16
EPT = E // NSUB
CHUNK = 80
NCHUNK = EPT // CHUNK
NPASS = 2
RP = 5120
NP = NPASS * RP
ACCR = RP + 8
RPT = RP // NSUB
ZR = 32
LSUB = 16
BM = 1000

_MESH = plsc.VectorSubcoreMesh(core_axis_name="c", subcore_axis_name="s")


def _rebase(dst_v, adj_v, j, base):
    for c in range(CHUNK // LSUB):
        d = dst_v[j, pl.ds(c * LSUB, LSUB)] - base
        ok = (d >= 0) & (d < RP)
        adj_v[0, pl.ds(c * LSUB, LSUB)] = jnp.where(ok, d, jnp.int32(RP))


def _zero_acc(zbuf_v, acc_sh, row0):
    for z in range(RPT // ZR):
        pltpu.sync_copy(zbuf_v, acc_sh.at[pl.ds(row0 + z * ZR, ZR)])


def _segsum_body(g_hbm, src_hbm, dst_hbm, sums_hbm,
                 src_v, dst_v, adj_v, buf_v, zbuf_v,
                 acc_sh, sem_g):
    cid = lax.axis_index("c")
    sid = lax.axis_index("s")
    row0 = sid * RPT

    pltpu.sync_copy(src_hbm.at[sid], src_v)
    pltpu.sync_copy(dst_hbm.at[sid], dst_v)

    @pl.loop(0, ZR)
    def _zero(i):
        for c in range(DH // LSUB):
            zbuf_v[i, pl.ds(c * LSUB, LSUB)] = jnp.zeros((LSUB,), jnp.float32)

    gtab = g_hbm.at[cid]

    for r in range(NPASS):
        base = jnp.int32(r * RP)
        _zero_acc(zbuf_v, acc_sh, row0)
        plsc.subcore_barrier()

        @pl.loop(0, NCHUNK)
        def _edges(j):
            cp = pltpu.async_copy(gtab.at[src_v.at[j]], buf_v, sem_g)
            _rebase(dst_v, adj_v, j, base)
            cp.wait()
            pltpu.sync_copy(buf_v, acc_sh.at[adj_v.at[0]], add=True)

        plsc.subcore_barrier()
        pltpu.sync_copy(acc_sh.at[pl.ds(row0, RPT)],
                        sums_hbm.at[cid, pl.ds(r * RP + row0, RPT)])


def _segsum(g, src3, dst3):
    f = pl.kernel(
        _segsum_body,
        out_type=[jax.ShapeDtypeStruct((NCORE, NP, DH), jnp.float32)],
        mesh=_MESH,
        scratch_types=[
            pltpu.VMEM((NCHUNK, CHUNK), jnp.int32),
            pltpu.VMEM((NCHUNK, CHUNK), jnp.int32),
            pltpu.VMEM((8, CHUNK), jnp.int32),
            pltpu.VMEM((CHUNK, DH), jnp.float32),
            pltpu.VMEM((ZR, DH), jnp.float32),
            pltpu.VMEM_SHARED((ACCR, DH), jnp.float32),
            pltpu.SemaphoreType.DMA,
        ],
    )
    return f(g, src3, dst3)[0]


def _degree_body(dst_hbm, cnt_hbm, dst_v, adj_v, ones_v, zbuf_v, acc_sh):
    cid = lax.axis_index("c")
    sid = lax.axis_index("s")
    row0 = sid * RPT

    pltpu.sync_copy(dst_hbm.at[sid], dst_v)

    @pl.loop(0, ZR)
    def _zero(i):
        for c in range(DH // LSUB):
            zbuf_v[i, pl.ds(c * LSUB, LSUB)] = jnp.zeros((LSUB,), jnp.float32)

    @pl.loop(0, CHUNK)
    def _ones(i):
        for c in range(DH // LSUB):
            ones_v[i, pl.ds(c * LSUB, LSUB)] = jnp.ones((LSUB,), jnp.float32)

    base = cid * RP
    _zero_acc(zbuf_v, acc_sh, row0)
    plsc.subcore_barrier()

    @pl.loop(0, NCHUNK)
    def _edges(j):
        _rebase(dst_v, adj_v, j, base)
        pltpu.sync_copy(ones_v, acc_sh.at[adj_v.at[0]], add=True)

    plsc.subcore_barrier()
    pltpu.sync_copy(acc_sh.at[pl.ds(row0, RPT)],
                    cnt_hbm.at[pl.ds(cid * RP + row0, RPT)])


def _degree(dst3):
    f = pl.kernel(
        _degree_body,
        out_type=[jax.ShapeDtypeStruct((NP, DH), jnp.float32)],
        mesh=_MESH,
        scratch_types=[
            pltpu.VMEM((NCHUNK, CHUNK), jnp.int32),
            pltpu.VMEM((8, CHUNK), jnp.int32),
            pltpu.VMEM((CHUNK, DH), jnp.float32),
            pltpu.VMEM((ZR, DH), jnp.float32),
            pltpu.VMEM_SHARED((ACCR, DH), jnp.float32),
        ],
    )
    return f(dst3)[0]


def _transform_body(x_ref, wlt_ref, g_ref):
    g_ref[0] = jnp.dot(x_ref[...], wlt_ref[0],
                       preferred_element_type=jnp.float32)


def _transform(x, wlt2):
    return pl.pallas_call(
        _transform_body,
        grid=(NCORE, N // BM),
        in_specs=[
            pl.BlockSpec((BM, D), lambda c, m: (m, 0)),
            pl.BlockSpec((1, D, DH), lambda c, m: (c, 0, 0)),
        ],
        out_specs=pl.BlockSpec((1, BM, DH), lambda c, m: (c, m, 0)),
        out_shape=jax.ShapeDtypeStruct((NCORE, N, DH), jnp.float32),
    )(x, wlt2)


def _combine_body(s0_ref, s1_ref, cnt_ref, x_ref, b_ref, wrt_ref,
                  o_ref, *, relu):
    inv = 1.0 / jnp.maximum(cnt_ref[:, :1], 1.0)
    mean = jnp.concatenate([s0_ref[0] * inv, s1_ref[0] * inv], axis=1)
    acc = mean + jnp.dot(x_ref[...], wrt_ref[...],
                         preferred_element_type=jnp.float32)
    acc = acc + b_ref[...]
    o_ref[...] = jnp.maximum(acc, 0.0) if relu else acc


def _combine(s, cnt, x, b, wrt, relu):
    return pl.pallas_call(
        functools.partial(_combine_body, relu=relu),
        grid=(N // BM,),
        in_specs=[
            pl.BlockSpec((1, BM, DH), lambda m: (0, m, 0)),
            pl.BlockSpec((1, BM, DH), lambda m: (1, m, 0)),
            pl.BlockSpec((BM, DH), lambda m: (m, 0)),
            pl.BlockSpec((BM, D), lambda m: (m, 0)),
            pl.BlockSpec((1, D), lambda m: (0, 0)),
            pl.BlockSpec((D, D), lambda m: (0, 0)),
        ],
        out_specs=pl.BlockSpec((BM, D), lambda m: (m, 0)),
        out_shape=jax.ShapeDtypeStruct((N, D), jnp.float32),
    )(s, s, cnt, x, b, wrt)


def kernel(x, edge_index, W1_l, b1, W1_r, W2_l, b2, W2_r):
    src3 = edge_index[0].reshape(NSUB, NCHUNK, CHUNK)
    dst3 = edge_index[1].reshape(NSUB, NCHUNK, CHUNK)

    w1lt2 = W1_l.T.reshape(D, NCORE, DH).transpose(1, 0, 2)
    w2lt2 = W2_l.T.reshape(D, NCORE, DH).transpose(1, 0, 2)

    cnt = _degree(dst3)
    g1 = _transform(x, w1lt2)
    s1 = _segsum(g1, src3, dst3)
    h = _combine(s1, cnt, x, b1.reshape(1, D), W1_r.T, relu=True)

    g2 = _transform(h, w2lt2)
    s2 = _segsum(g2, src3, dst3)
    out = _combine(s2, cnt, h, b2.reshape(1, D), W2_r.T, relu=False)
    return out


# speedup vs baseline: 3.6828x; 1.3843x over previous
import functools

import jax
import jax.numpy as jnp
from jax import lax
from jax.experimental import pallas as pl
from jax.experimental.pallas import tpu as pltpu
from jax.experimental.pallas import tpu_sc as plsc

N = 10000
E = 160000
D = 256
DH = 128
NCORE = 2
NSUB = 16
EPT = E // NSUB
CHUNK = 80
NCHUNK = EPT // CHUNK
NPASS = 2
RP = 5120
NP = NPASS * RP
ACCR = RP + 8
RPT = RP // NSUB
ZR = 16
LSUB = 16
BM = 1000

_MESH = plsc.VectorSubcoreMesh(core_axis_name="c", subcore_axis_name="s")


def _rebase(dst_v, adj_v, j, base):
    for c in range(CHUNK // LSUB):
        d = dst_v[j, pl.ds(c * LSUB, LSUB)] - base
        ok = (d >= 0) & (d < RP)
        adj_v[0, pl.ds(c * LSUB, LSUB)] = jnp.where(ok, d, jnp.int32(RP))


def _zero_acc(zbuf_v, acc_sh, row0):
    for z in range(RPT // ZR):
        pltpu.sync_copy(zbuf_v, acc_sh.at[pl.ds(row0 + z * ZR, ZR)])


def _segsum_body(g_hbm, src_hbm, dst_hbm, sums_hbm,
                 src_v, dst_v, adj_v, buf0_v, buf1_v, zbuf_v,
                 acc_sh, sem0, sem1):
    cid = lax.axis_index("c")
    sid = lax.axis_index("s")
    row0 = sid * RPT

    pltpu.sync_copy(src_hbm.at[sid], src_v)
    pltpu.sync_copy(dst_hbm.at[sid], dst_v)

    @pl.loop(0, ZR)
    def _zero(i):
        for c in range(DH // LSUB):
            zbuf_v[i, pl.ds(c * LSUB, LSUB)] = jnp.zeros((LSUB,), jnp.float32)

    gtab = g_hbm.at[cid]
    bufs = (buf0_v, buf1_v)
    sems = (sem0, sem1)

    for r in range(NPASS):
        base = jnp.int32(r * RP)
        _zero_acc(zbuf_v, acc_sh, row0)
        plsc.subcore_barrier()

        pltpu.async_copy(gtab.at[src_v.at[0]], buf0_v, sem0)

        @pl.loop(0, NCHUNK // 2)
        def _pair(p):
            j0 = 2 * p
            for b in range(2):
                j = j0 + b
                nxt = j + 1

                @pl.when(nxt < NCHUNK)
                def _start():
                    pltpu.async_copy(gtab.at[src_v.at[nxt]],
                                     bufs[1 - b], sems[1 - b])

                pltpu.make_async_copy(gtab.at[src_v.at[j]],
                                      bufs[b], sems[b]).wait()
                _rebase(dst_v, adj_v, j, base)
                pltpu.sync_copy(bufs[b], acc_sh.at[adj_v.at[0]], add=True)

        if NCHUNK % 2:
            last = NCHUNK - 1
            lb = last % 2
            pltpu.make_async_copy(gtab.at[src_v.at[last]],
                                  bufs[lb], sems[lb]).wait()
            _rebase(dst_v, adj_v, last, base)
            pltpu.sync_copy(bufs[lb], acc_sh.at[adj_v.at[0]], add=True)

        plsc.subcore_barrier()
        pltpu.sync_copy(acc_sh.at[pl.ds(row0, RPT)],
                        sums_hbm.at[cid, pl.ds(r * RP + row0, RPT)])


def _segsum(g, src3, dst3):
    f = pl.kernel(
        _segsum_body,
        out_type=[jax.ShapeDtypeStruct((NCORE, NP, DH), jnp.float32)],
        mesh=_MESH,
        scratch_types=[
            pltpu.VMEM((NCHUNK, CHUNK), jnp.int32),
            pltpu.VMEM((NCHUNK, CHUNK), jnp.int32),
            pltpu.VMEM((8, CHUNK), jnp.int32),
            pltpu.VMEM((CHUNK, DH), jnp.float32),
            pltpu.VMEM((CHUNK, DH), jnp.float32),
            pltpu.VMEM((ZR, DH), jnp.float32),
            pltpu.VMEM_SHARED((ACCR, DH), jnp.float32),
            pltpu.SemaphoreType.DMA,
            pltpu.SemaphoreType.DMA,
        ],
    )
    return f(g, src3, dst3)[0]


def _degree_body(dst_hbm, cnt_hbm, dst_v, adj_v, ones_v, zbuf_v, acc_sh):
    cid = lax.axis_index("c")
    sid = lax.axis_index("s")
    row0 = sid * RPT

    pltpu.sync_copy(dst_hbm.at[sid], dst_v)

    @pl.loop(0, ZR)
    def _zero(i):
        for c in range(DH // LSUB):
            zbuf_v[i, pl.ds(c * LSUB, LSUB)] = jnp.zeros((LSUB,), jnp.float32)

    @pl.loop(0, CHUNK)
    def _ones(i):
        for c in range(DH // LSUB):
            ones_v[i, pl.ds(c * LSUB, LSUB)] = jnp.ones((LSUB,), jnp.float32)

    base = cid * RP
    _zero_acc(zbuf_v, acc_sh, row0)
    plsc.subcore_barrier()

    @pl.loop(0, NCHUNK)
    def _edges(j):
        _rebase(dst_v, adj_v, j, base)
        pltpu.sync_copy(ones_v, acc_sh.at[adj_v.at[0]], add=True)

    plsc.subcore_barrier()
    pltpu.sync_copy(acc_sh.at[pl.ds(row0, RPT)],
                    cnt_hbm.at[pl.ds(cid * RP + row0, RPT)])


def _degree(dst3):
    f = pl.kernel(
        _degree_body,
        out_type=[jax.ShapeDtypeStruct((NP, DH), jnp.float32)],
        mesh=_MESH,
        scratch_types=[
            pltpu.VMEM((NCHUNK, CHUNK), jnp.int32),
            pltpu.VMEM((8, CHUNK), jnp.int32),
            pltpu.VMEM((CHUNK, DH), jnp.float32),
            pltpu.VMEM((ZR, DH), jnp.float32),
            pltpu.VMEM_SHARED((ACCR, DH), jnp.float32),
        ],
    )
    return f(dst3)[0]


def _transform_body(x_ref, wlt_ref, g_ref):
    g_ref[0] = jnp.dot(x_ref[...], wlt_ref[0],
                       preferred_element_type=jnp.float32)


def _transform(x, wlt2):
    return pl.pallas_call(
        _transform_body,
        grid=(NCORE, N // BM),
        in_specs=[
            pl.BlockSpec((BM, D), lambda c, m: (m, 0)),
            pl.BlockSpec((1, D, DH), lambda c, m: (c, 0, 0)),
        ],
        out_specs=pl.BlockSpec((1, BM, DH), lambda c, m: (c, m, 0)),
        out_shape=jax.ShapeDtypeStruct((NCORE, N, DH), jnp.float32),
    )(x, wlt2)


def _combine_body(s0_ref, s1_ref, cnt_ref, x_ref, b_ref, wrt_ref,
                  o_ref, *, relu):
    inv = 1.0 / jnp.maximum(cnt_ref[:, :1], 1.0)
    mean = jnp.concatenate([s0_ref[0] * inv, s1_ref[0] * inv], axis=1)
    acc = mean + jnp.dot(x_ref[...], wrt_ref[...],
                         preferred_element_type=jnp.float32)
    acc = acc + b_ref[...]
    o_ref[...] = jnp.maximum(acc, 0.0) if relu else acc


def _combine(s, cnt, x, b, wrt, relu):
    return pl.pallas_call(
        functools.partial(_combine_body, relu=relu),
        grid=(N // BM,),
        in_specs=[
            pl.BlockSpec((1, BM, DH), lambda m: (0, m, 0)),
            pl.BlockSpec((1, BM, DH), lambda m: (1, m, 0)),
            pl.BlockSpec((BM, DH), lambda m: (m, 0)),
            pl.BlockSpec((BM, D), lambda m: (m, 0)),
            pl.BlockSpec((1, D), lambda m: (0, 0)),
            pl.BlockSpec((D, D), lambda m: (0, 0)),
        ],
        out_specs=pl.BlockSpec((BM, D), lambda m: (m, 0)),
        out_shape=jax.ShapeDtypeStruct((N, D), jnp.float32),
    )(s, s, cnt, x, b, wrt)


def kernel(x, edge_index, W1_l, b1, W1_r, W2_l, b2, W2_r):
    src3 = edge_index[0].reshape(NSUB, NCHUNK, CHUNK)
    dst3 = edge_index[1].reshape(NSUB, NCHUNK, CHUNK)

    w1lt2 = W1_l.T.reshape(D, NCORE, DH).transpose(1, 0, 2)
    w2lt2 = W2_l.T.reshape(D, NCORE, DH).transpose(1, 0, 2)

    cnt = _degree(dst3)
    g1 = _transform(x, w1lt2)
    s1 = _segsum(g1, src3, dst3)
    h = _combine(s1, cnt, x, b1.reshape(1, D), W1_r.T, relu=True)

    g2 = _transform(h, w2lt2)
    s2 = _segsum(g2, src3, dst3)
    out = _combine(s2, cnt, h, b2.reshape(1, D), W2_r.T, relu=False)
    return out
